# Initial kernel scaffold; baseline (speedup 1.0000x reference)
import jax
import jax.numpy as jnp
from jax.experimental import pallas as pl


def kernel(x, edge_index, lambda_max, W0_0, W0_1, b0, W1_0, W1_1, b1, W2_0, W2_1, b2, Wf, bf):
    raise NotImplementedError("write your pallas kernel here")


# speedup vs baseline: 26.5820x; 26.5820x over previous
import functools

import jax
import jax.numpy as jnp
from jax import lax
from jax.experimental import pallas as pl
from jax.experimental.pallas import tpu as pltpu
from jax.experimental.pallas import tpu_sc as plsc

_NC = 2
_NS = 16
_NW = _NC * _NS
_LN = 128
_WR = 8
_SC_PARAMS = pltpu.CompilerParams(use_tc_tiling_on_sc=False)


def _prep_body(row_ref, col_ref, gidx_ref, sidx_ref, didx_ref, *, n, np_, rows,
               rows_pad):
    row = row_ref[...]
    col = col_ref[...]
    keep = row != col
    eid = lax.broadcasted_iota(jnp.int32, (rows, _LN), 0) * _LN + \
        lax.broadcasted_iota(jnp.int32, (rows, _LN), 1)
    dummy = n + lax.rem(eid, jnp.int32(np_ - n))
    sidx_ref[:rows] = jnp.where(keep, col, dummy)
    didx_ref[:rows] = jnp.where(keep, row, dummy)
    gidx_ref[:rows] = row
    if rows_pad > rows:
        pe = lax.broadcasted_iota(jnp.int32, (rows_pad - rows, _LN), 0) * _LN \
            + lax.broadcasted_iota(jnp.int32, (rows_pad - rows, _LN), 1)
        pad_dummy = n + lax.rem(pe, jnp.int32(np_ - n))
        gidx_ref[rows:] = lax.rem(pe, jnp.int32(n))
        sidx_ref[rows:] = pad_dummy
        didx_ref[rows:] = pad_dummy


def _proj0_body(x_ref, w0_ref, w1_ref, b_ref, lam_ref, yp_ref, z_ref):
    lam = lam_ref[0, 0]
    diag = 2.0 / lam - 1.0
    x = x_ref[...]
    w1 = w1_ref[...]
    w0eff = w0_ref[...] + diag * w1
    yp_ref[...] = jnp.dot(x, w0eff, preferred_element_type=jnp.float32) \
        + b_ref[...]
    z_ref[...] = jnp.dot(x, w1, preferred_element_type=jnp.float32)


def _dis_body(deg_ref, z_ref, dis_ref, zt_ref, *, n, np_):
    deg = deg_ref[:np_] + deg_ref[np_:]
    dis = jnp.where(deg > 0.0, lax.rsqrt(jnp.maximum(deg, 1e-30)), 0.0)
    dis_n = dis[:n]
    dis_ref[...] = dis_n
    zt_ref[...] = dis_n[:, 0:1] * z_ref[...]


def _mid_body(acc_ref, dis_ref, yp_ref, w0_ref, w1_ref, b_ref, lam_ref,
              yp2_ref, zt2_ref, *, n, np_):
    lam = lam_ref[0, 0]
    c = 2.0 / lam
    diag = c - 1.0
    accsum = acc_ref[:n] + acc_ref[np_:np_ + n]
    dis_c = dis_ref[...][:, 0:1]
    h = jnp.maximum(yp_ref[...] - c * dis_c * accsum, 0.0)
    w1 = w1_ref[...]
    w0eff = w0_ref[...] + diag * w1
    yp2_ref[...] = jnp.dot(h, w0eff, preferred_element_type=jnp.float32) \
        + b_ref[...]
    zt2_ref[...] = dis_c * jnp.dot(h, w1, preferred_element_type=jnp.float32)


def _final_body(acc_ref, dis_ref, yp_ref, wf_ref, bf_ref, lam_ref, out_ref,
                *, n, np_):
    lam = lam_ref[0, 0]
    c = 2.0 / lam
    accsum = acc_ref[:n] + acc_ref[np_:np_ + n]
    dis_c = dis_ref[...][:, 0:1]
    h = jnp.maximum(yp_ref[...] - c * dis_c * accsum, 0.0)
    out_ref[...] = jnp.dot(h, wf_ref[...], preferred_element_type=jnp.float32) \
        + bf_ref[...]


def _sc_deg_body(didx_hbm, ones_hbm, zeros_hbm, out_hbm, didx_v, ones_v, acc,
                 sem, *, np_, per_sub, rows_per_w, nwin):
    c = lax.axis_index("c")
    s = lax.axis_index("s")
    wid = c * _NS + s
    pltpu.sync_copy(zeros_hbm, acc.at[pl.ds(s * per_sub, per_sub)])
    pltpu.sync_copy(ones_hbm, ones_v)
    plsc.subcore_barrier()
    base = wid * rows_per_w

    def win(i, carry):
        pltpu.async_copy(didx_hbm.at[pl.ds(base + i * _WR, _WR)], didx_v,
                         sem).wait()
        for j in range(_WR):
            pltpu.sync_copy(ones_v, acc.at[didx_v.at[j]], add=True)
        return carry

    lax.fori_loop(0, nwin, win, 0)
    plsc.subcore_barrier()
    pltpu.sync_copy(acc.at[pl.ds(s * per_sub, per_sub)],
                    out_hbm.at[pl.ds((c * _NS + s) * per_sub, per_sub)])


def _sc_spmv_body(gidx_hbm, sidx_hbm, zt_hbm, zeros_hbm, out_hbm, gidx_v,
                  sidx_v, rows_v, acc, sem, *, n, np_, per_sub,
                  rows_per_w, nwin, h):
    c = lax.axis_index("c")
    s = lax.axis_index("s")
    wid = c * _NS + s
    pltpu.sync_copy(zeros_hbm, acc.at[pl.ds(s * per_sub, per_sub)])
    plsc.subcore_barrier()
    base = wid * rows_per_w

    def win(i, carry):
        pltpu.async_copy(gidx_hbm.at[pl.ds(base + i * _WR, _WR)], gidx_v,
                         sem).wait()
        pltpu.async_copy(sidx_hbm.at[pl.ds(base + i * _WR, _WR)], sidx_v,
                         sem).wait()
        descs = []
        for j in range(_WR):
            descs.append(
                pltpu.async_copy(zt_hbm.at[gidx_v.at[j]],
                                 rows_v.at[pl.ds(j * _LN, _LN)], sem))
        for d in descs:
            d.wait()
        for j in range(_WR):
            pltpu.sync_copy(rows_v.at[pl.ds(j * _LN, _LN)],
                            acc.at[sidx_v.at[j]], add=True)
        return carry

    lax.fori_loop(0, nwin, win, 0)
    plsc.subcore_barrier()
    pltpu.sync_copy(acc.at[pl.ds(s * per_sub, per_sub)],
                    out_hbm.at[pl.ds((c * _NS + s) * per_sub, per_sub)])


def kernel(x, edge_index, lambda_max, W0_0, W0_1, b0, W1_0, W1_1, b1, W2_0,
           W2_1, b2, Wf, bf):
    n, f_in = x.shape
    e = edge_index.shape[1]
    h = W0_0.shape[1]
    out_w = Wf.shape[1]
    f32 = jnp.float32

    rows = e // _LN
    rows_pad = ((rows + _NW * _WR - 1) // (_NW * _WR)) * (_NW * _WR)
    rows_per_w = rows_pad // _NW
    nwin = rows_per_w // _WR
    np_ = ((n + 2048 + _LN - 1) // _LN) * _LN
    per_sub = np_ // _NS

    row2 = edge_index[0].reshape(rows, _LN)
    col2 = edge_index[1].reshape(rows, _LN)
    lam2 = lambda_max.reshape(1, 1).astype(f32)
    zeros_h = jnp.zeros((per_sub, h), f32)
    zeros_d = jnp.zeros((per_sub, 8), f32)
    ones_d = jnp.ones((_LN, 8), f32)
    b0r = b0.reshape(1, h)
    b1r = b1.reshape(1, h)
    b2r = b2.reshape(1, h)
    bfr = bf.reshape(1, out_w)

    gidx, sidx, didx = pl.pallas_call(
        functools.partial(_prep_body, n=n, np_=np_, rows=rows,
                          rows_pad=rows_pad),
        out_shape=[jax.ShapeDtypeStruct((rows_pad, _LN), jnp.int32)] * 3,
    )(row2, col2)

    mesh = plsc.VectorSubcoreMesh(core_axis_name="c", subcore_axis_name="s")
    deg = pl.kernel(
        functools.partial(_sc_deg_body, np_=np_, per_sub=per_sub,
                          rows_per_w=rows_per_w, nwin=nwin),
        out_type=jax.ShapeDtypeStruct((_NC * np_, 8), f32),
        mesh=mesh,
        compiler_params=_SC_PARAMS,
        scratch_types=[
            pltpu.VMEM((_WR, _LN), jnp.int32),
            pltpu.VMEM((_LN, 8), f32),
            pltpu.VMEM_SHARED((np_, 8), f32),
            pltpu.SemaphoreType.DMA,
        ],
    )(didx, ones_d, zeros_d)

    y1p, z1 = pl.pallas_call(
        _proj0_body,
        out_shape=[jax.ShapeDtypeStruct((n, h), f32)] * 2,
    )(x, W0_0, W0_1, b0r, lam2)

    dis8, zt1 = pl.pallas_call(
        functools.partial(_dis_body, n=n, np_=np_),
        out_shape=[jax.ShapeDtypeStruct((n, 8), f32),
                   jax.ShapeDtypeStruct((n, h), f32)],
    )(deg, z1)

    spmv = pl.kernel(
        functools.partial(_sc_spmv_body, n=n, np_=np_, per_sub=per_sub,
                          rows_per_w=rows_per_w, nwin=nwin, h=h),
        out_type=jax.ShapeDtypeStruct((_NC * np_, h), f32),
        mesh=mesh,
        compiler_params=_SC_PARAMS,
        scratch_types=[
            pltpu.VMEM((_WR, _LN), jnp.int32),
            pltpu.VMEM((_WR, _LN), jnp.int32),
            pltpu.VMEM((_WR * _LN, h), f32),
            pltpu.VMEM_SHARED((np_, h), f32),
            pltpu.SemaphoreType.DMA,
        ],
    )

    mid = functools.partial(_mid_body, n=n, np_=np_)

    acc1 = spmv(gidx, sidx, zt1, zeros_h)
    y2p, zt2 = pl.pallas_call(
        mid,
        out_shape=[jax.ShapeDtypeStruct((n, h), f32)] * 2,
    )(acc1, dis8, y1p, W1_0, W1_1, b1r, lam2)

    acc2 = spmv(gidx, sidx, zt2, zeros_h)
    y3p, zt3 = pl.pallas_call(
        mid,
        out_shape=[jax.ShapeDtypeStruct((n, h), f32)] * 2,
    )(acc2, dis8, y2p, W2_0, W2_1, b2r, lam2)

    acc3 = spmv(gidx, sidx, zt3, zeros_h)
    out = pl.pallas_call(
        functools.partial(_final_body, n=n, np_=np_),
        out_shape=jax.ShapeDtypeStruct((n, out_w), f32),
    )(acc3, dis8, y3p, Wf, bfr, lam2)
    return out


# speedup vs baseline: 30.8538x; 1.1607x over previous
import functools

import jax
import jax.numpy as jnp
from jax import lax
from jax.experimental import pallas as pl
from jax.experimental.pallas import tpu as pltpu
from jax.experimental.pallas import tpu_sc as plsc

_NC = 2
_NS = 16
_NW = _NC * _NS
_LN = 128
_WR = 8
_SC_PARAMS = pltpu.CompilerParams(use_tc_tiling_on_sc=False)


def _prep_body(row_ref, col_ref, gidx_ref, sidx_ref, didx_ref, *, n, np_, rows,
               rows_pad):
    row = row_ref[...]
    col = col_ref[...]
    keep = row != col
    eid = lax.broadcasted_iota(jnp.int32, (rows, _LN), 0) * _LN + \
        lax.broadcasted_iota(jnp.int32, (rows, _LN), 1)
    dummy = n + lax.rem(eid, jnp.int32(np_ - n))
    sidx_ref[:rows] = jnp.where(keep, col, dummy)
    didx_ref[:rows] = jnp.where(keep, row, dummy)
    gidx_ref[:rows] = row
    if rows_pad > rows:
        pe = lax.broadcasted_iota(jnp.int32, (rows_pad - rows, _LN), 0) * _LN \
            + lax.broadcasted_iota(jnp.int32, (rows_pad - rows, _LN), 1)
        pad_dummy = n + lax.rem(pe, jnp.int32(np_ - n))
        gidx_ref[rows:] = lax.rem(pe, jnp.int32(n))
        sidx_ref[rows:] = pad_dummy
        didx_ref[rows:] = pad_dummy


def _proj0_body(x_ref, w0_ref, w1_ref, b_ref, lam_ref, yp_ref, z_ref):
    lam = lam_ref[0, 0]
    diag = 2.0 / lam - 1.0
    x = x_ref[...]
    w1 = w1_ref[...]
    w0eff = w0_ref[...] + diag * w1
    yp_ref[...] = jnp.dot(x, w0eff, preferred_element_type=jnp.float32) \
        + b_ref[...]
    z_ref[...] = jnp.dot(x, w1, preferred_element_type=jnp.float32)


def _dis_body(deg_ref, z_ref, dis_ref, zt_ref, *, n, np_):
    deg = deg_ref[:np_] + deg_ref[np_:]
    dis = jnp.where(deg > 0.0, lax.rsqrt(jnp.maximum(deg, 1e-30)), 0.0)
    dis_n = dis[:n]
    dis_ref[...] = dis_n
    zt_ref[...] = dis_n[:, 0:1] * z_ref[...]


def _mid_body(acc_ref, dis_ref, yp_ref, w0_ref, w1_ref, b_ref, lam_ref,
              yp2_ref, zt2_ref, *, n, np_):
    lam = lam_ref[0, 0]
    c = 2.0 / lam
    diag = c - 1.0
    accsum = acc_ref[:n] + acc_ref[np_:np_ + n]
    dis_c = dis_ref[...][:, 0:1]
    h = jnp.maximum(yp_ref[...] - c * dis_c * accsum, 0.0)
    w1 = w1_ref[...]
    w0eff = w0_ref[...] + diag * w1
    yp2_ref[...] = jnp.dot(h, w0eff, preferred_element_type=jnp.float32) \
        + b_ref[...]
    zt2_ref[...] = dis_c * jnp.dot(h, w1, preferred_element_type=jnp.float32)


def _final_body(acc_ref, dis_ref, yp_ref, wf_ref, bf_ref, lam_ref, out_ref,
                *, n, np_):
    lam = lam_ref[0, 0]
    c = 2.0 / lam
    accsum = acc_ref[:n] + acc_ref[np_:np_ + n]
    dis_c = dis_ref[...][:, 0:1]
    h = jnp.maximum(yp_ref[...] - c * dis_c * accsum, 0.0)
    out_ref[...] = jnp.dot(h, wf_ref[...], preferred_element_type=jnp.float32) \
        + bf_ref[...]


def _sc_deg_body(didx_hbm, ones_hbm, zeros_hbm, out_hbm, didx_all, ones_v,
                 acc, ssa, *, np_, per_sub, nwin):
    c = lax.axis_index("c")
    s = lax.axis_index("s")
    wid = c * _NS + s
    pltpu.sync_copy(zeros_hbm, acc.at[pl.ds(s * per_sub, per_sub)])
    pltpu.sync_copy(ones_hbm, ones_v)
    pltpu.sync_copy(didx_hbm.at[pl.ds(wid * nwin, nwin)], didx_all)
    plsc.subcore_barrier()

    def win(i, carry):
        sub = didx_all.at[i]
        descs = []
        for j in range(_WR):
            descs.append(
                pltpu.async_copy(ones_v, acc.at[sub.at[j]], ssa, add=True))
        for d in descs:
            d.wait()
        return carry

    lax.fori_loop(0, nwin, win, 0)
    plsc.subcore_barrier()
    pltpu.sync_copy(acc.at[pl.ds(s * per_sub, per_sub)],
                    out_hbm.at[pl.ds((c * _NS + s) * per_sub, per_sub)])


def _sc_spmv_body(gidx_hbm, sidx_hbm, zt_hbm, zeros_hbm, out_hbm, gidx_all,
                  sidx_all, rows_v, acc, gsa, ssa, *, n, np_,
                  per_sub, nwin, h):
    c = lax.axis_index("c")
    s = lax.axis_index("s")
    wid = c * _NS + s
    pltpu.sync_copy(zeros_hbm, acc.at[pl.ds(s * per_sub, per_sub)])
    pltpu.sync_copy(gidx_hbm.at[pl.ds(wid * nwin, nwin)], gidx_all)
    pltpu.sync_copy(sidx_hbm.at[pl.ds(wid * nwin, nwin)], sidx_all)

    plsc.subcore_barrier()

    def win(i, carry):
        gsub = gidx_all.at[i]
        ssub = sidx_all.at[i]
        gds = []
        for j in range(_WR):
            gds.append(
                pltpu.async_copy(zt_hbm.at[gsub.at[j]],
                                 rows_v.at[pl.ds(j * _LN, _LN)], gsa))
        for d in gds:
            d.wait()
        sds = []
        for j in range(_WR):
            sds.append(
                pltpu.async_copy(rows_v.at[pl.ds(j * _LN, _LN)],
                                 acc.at[ssub.at[j]], ssa, add=True))
        for d in sds:
            d.wait()
        return carry

    lax.fori_loop(0, nwin, win, 0)
    plsc.subcore_barrier()
    pltpu.sync_copy(acc.at[pl.ds(s * per_sub, per_sub)],
                    out_hbm.at[pl.ds((c * _NS + s) * per_sub, per_sub)])


def kernel(x, edge_index, lambda_max, W0_0, W0_1, b0, W1_0, W1_1, b1, W2_0,
           W2_1, b2, Wf, bf):
    n, f_in = x.shape
    e = edge_index.shape[1]
    h = W0_0.shape[1]
    out_w = Wf.shape[1]
    f32 = jnp.float32

    rows = e // _LN
    rows_pad = ((rows + _NW * _WR - 1) // (_NW * _WR)) * (_NW * _WR)
    rows_per_w = rows_pad // _NW
    nwin = rows_per_w // _WR
    np_ = ((n + 2048 + _LN - 1) // _LN) * _LN
    per_sub = np_ // _NS

    row2 = edge_index[0].reshape(rows, _LN)
    col2 = edge_index[1].reshape(rows, _LN)
    lam2 = lambda_max.reshape(1, 1).astype(f32)
    zeros_h = jnp.zeros((per_sub, h), f32)
    zeros_d = jnp.zeros((per_sub, 8), f32)
    ones_d = jnp.ones((_LN, 8), f32)
    b0r = b0.reshape(1, h)
    b1r = b1.reshape(1, h)
    b2r = b2.reshape(1, h)
    bfr = bf.reshape(1, out_w)

    gidx, sidx, didx = pl.pallas_call(
        functools.partial(_prep_body, n=n, np_=np_, rows=rows,
                          rows_pad=rows_pad),
        out_shape=[jax.ShapeDtypeStruct((rows_pad, _LN), jnp.int32)] * 3,
    )(row2, col2)

    gidx3 = gidx.reshape(rows_pad // _WR, _WR, _LN)
    sidx3 = sidx.reshape(rows_pad // _WR, _WR, _LN)
    didx3 = didx.reshape(rows_pad // _WR, _WR, _LN)
    mesh = plsc.VectorSubcoreMesh(core_axis_name="c", subcore_axis_name="s")
    deg = pl.kernel(
        functools.partial(_sc_deg_body, np_=np_, per_sub=per_sub, nwin=nwin),
        out_type=jax.ShapeDtypeStruct((_NC * np_, 8), f32),
        mesh=mesh,
        compiler_params=_SC_PARAMS,
        scratch_types=[
            pltpu.VMEM((nwin, _WR, _LN), jnp.int32),
            pltpu.VMEM((_LN, 8), f32),
            pltpu.VMEM_SHARED((np_, 8), f32),
            pltpu.SemaphoreType.DMA,
        ],
    )(didx3, ones_d, zeros_d)

    y1p, z1 = pl.pallas_call(
        _proj0_body,
        out_shape=[jax.ShapeDtypeStruct((n, h), f32)] * 2,
    )(x, W0_0, W0_1, b0r, lam2)

    dis8, zt1 = pl.pallas_call(
        functools.partial(_dis_body, n=n, np_=np_),
        out_shape=[jax.ShapeDtypeStruct((n, 8), f32),
                   jax.ShapeDtypeStruct((n, h), f32)],
    )(deg, z1)

    spmv = pl.kernel(
        functools.partial(_sc_spmv_body, n=n, np_=np_, per_sub=per_sub,
                          nwin=nwin, h=h),
        out_type=jax.ShapeDtypeStruct((_NC * np_, h), f32),
        mesh=mesh,
        compiler_params=_SC_PARAMS,
        scratch_types=[
            pltpu.VMEM((nwin, _WR, _LN), jnp.int32),
            pltpu.VMEM((nwin, _WR, _LN), jnp.int32),
            pltpu.VMEM((_WR * _LN, h), f32),
            pltpu.VMEM_SHARED((np_, h), f32),
            pltpu.SemaphoreType.DMA,
            pltpu.SemaphoreType.DMA,
        ],
    )

    mid = functools.partial(_mid_body, n=n, np_=np_)

    acc1 = spmv(gidx3, sidx3, zt1, zeros_h)
    y2p, zt2 = pl.pallas_call(
        mid,
        out_shape=[jax.ShapeDtypeStruct((n, h), f32)] * 2,
    )(acc1, dis8, y1p, W1_0, W1_1, b1r, lam2)

    acc2 = spmv(gidx3, sidx3, zt2, zeros_h)
    y3p, zt3 = pl.pallas_call(
        mid,
        out_shape=[jax.ShapeDtypeStruct((n, h), f32)] * 2,
    )(acc2, dis8, y2p, W2_0, W2_1, b2r, lam2)

    acc3 = spmv(gidx3, sidx3, zt3, zeros_h)
    out = pl.pallas_call(
        functools.partial(_final_body, n=n, np_=np_),
        out_shape=jax.ShapeDtypeStruct((n, out_w), f32),
    )(acc3, dis8, y3p, Wf, bfr, lam2)
    return out


# speedup vs baseline: 34.7620x; 1.1267x over previous
import functools

import jax
import jax.numpy as jnp
from jax import lax
from jax.experimental import pallas as pl
from jax.experimental.pallas import tpu as pltpu
from jax.experimental.pallas import tpu_sc as plsc

_NC = 2
_NS = 16
_NW = _NC * _NS
_LN = 128
_WR = 8
_SC_PARAMS = pltpu.CompilerParams(use_tc_tiling_on_sc=False)


def _prep_body(row_ref, col_ref, gidx_ref, sidx_ref, didx_ref, *, n, np_, rows,
               rows_pad):
    row = row_ref[...]
    col = col_ref[...]
    keep = row != col
    eid = lax.broadcasted_iota(jnp.int32, (rows, _LN), 0) * _LN + \
        lax.broadcasted_iota(jnp.int32, (rows, _LN), 1)
    dummy = n + lax.rem(eid, jnp.int32(np_ - n))
    sidx_ref[:rows] = jnp.where(keep, col, dummy)
    didx_ref[:rows] = jnp.where(keep, row, dummy)
    gidx_ref[:rows] = row
    if rows_pad > rows:
        pe = lax.broadcasted_iota(jnp.int32, (rows_pad - rows, _LN), 0) * _LN \
            + lax.broadcasted_iota(jnp.int32, (rows_pad - rows, _LN), 1)
        pad_dummy = n + lax.rem(pe, jnp.int32(np_ - n))
        gidx_ref[rows:] = lax.rem(pe, jnp.int32(n))
        sidx_ref[rows:] = pad_dummy
        didx_ref[rows:] = pad_dummy


def _proj0_body(x_ref, w0_ref, w1_ref, b_ref, lam_ref, yp_ref, z_ref):
    lam = lam_ref[0, 0]
    diag = 2.0 / lam - 1.0
    x = x_ref[...]
    w1 = w1_ref[...]
    w0eff = w0_ref[...] + diag * w1
    yp_ref[...] = jnp.dot(x, w0eff, preferred_element_type=jnp.float32) \
        + b_ref[...]
    z_ref[...] = jnp.dot(x, w1, preferred_element_type=jnp.float32)


def _dis_body(deg_ref, z_ref, dis_ref, zt_ref, *, n, np_):
    deg = deg_ref[:np_] + deg_ref[np_:]
    dis = jnp.where(deg > 0.0, lax.rsqrt(jnp.maximum(deg, 1e-30)), 0.0)
    dis_n = dis[:n]
    dis_ref[...] = dis_n
    zt_ref[...] = dis_n[:, 0:1] * z_ref[...]


def _mid_body(acc_ref, dis_ref, yp_ref, w0_ref, w1_ref, b_ref, lam_ref,
              yp2_ref, zt2_ref, *, n, np_):
    lam = lam_ref[0, 0]
    c = 2.0 / lam
    diag = c - 1.0
    accsum = acc_ref[:n] + acc_ref[np_:np_ + n]
    dis_c = dis_ref[...][:, 0:1]
    h = jnp.maximum(yp_ref[...] - c * dis_c * accsum, 0.0)
    w1 = w1_ref[...]
    w0eff = w0_ref[...] + diag * w1
    yp2_ref[...] = jnp.dot(h, w0eff, preferred_element_type=jnp.float32) \
        + b_ref[...]
    zt2_ref[...] = dis_c * jnp.dot(h, w1, preferred_element_type=jnp.float32)


def _final_body(acc_ref, dis_ref, yp_ref, wf_ref, bf_ref, lam_ref, out_ref,
                *, n, np_):
    lam = lam_ref[0, 0]
    c = 2.0 / lam
    accsum = acc_ref[:n] + acc_ref[np_:np_ + n]
    dis_c = dis_ref[...][:, 0:1]
    h = jnp.maximum(yp_ref[...] - c * dis_c * accsum, 0.0)
    out_ref[...] = jnp.dot(h, wf_ref[...], preferred_element_type=jnp.float32) \
        + bf_ref[...]


def _sc_deg_body(didx_hbm, ones_hbm, zeros_hbm, out_hbm, didx_all, ones_v,
                 acc, ssa, *, np_, per_sub, nwin):
    c = lax.axis_index("c")
    s = lax.axis_index("s")
    wid = c * _NS + s
    pltpu.sync_copy(zeros_hbm, acc.at[pl.ds(s * per_sub, per_sub)])
    pltpu.sync_copy(ones_hbm, ones_v)
    pltpu.sync_copy(didx_hbm.at[pl.ds(wid * nwin, nwin)], didx_all)
    plsc.subcore_barrier()

    def win(i, carry):
        sub_a = didx_all.at[2 * i]
        sub_b = didx_all.at[2 * i + 1]
        descs = []
        for j in range(_WR):
            descs.append(
                pltpu.async_copy(ones_v, acc.at[sub_a.at[j]], ssa, add=True))
        for j in range(_WR):
            descs.append(
                pltpu.async_copy(ones_v, acc.at[sub_b.at[j]], ssa, add=True))
        for d in descs:
            d.wait()
        return carry

    lax.fori_loop(0, nwin // 2, win, 0)
    plsc.subcore_barrier()
    pltpu.sync_copy(acc.at[pl.ds(s * per_sub, per_sub)],
                    out_hbm.at[pl.ds((c * _NS + s) * per_sub, per_sub)])


def _sc_spmv_body(gidx_hbm, sidx_hbm, zt_hbm, zeros_hbm, out_hbm, gidx_all,
                  sidx_all, rows_v, acc, gsa, ssa, *, n, np_,
                  per_sub, nwin, h):
    c = lax.axis_index("c")
    s = lax.axis_index("s")
    wid = c * _NS + s
    pltpu.sync_copy(zeros_hbm, acc.at[pl.ds(s * per_sub, per_sub)])
    pltpu.sync_copy(gidx_hbm.at[pl.ds(wid * nwin, nwin)], gidx_all)
    pltpu.sync_copy(sidx_hbm.at[pl.ds(wid * nwin, nwin)], sidx_all)

    def fire_g(w, half):
        sub = gidx_all.at[w]
        return [
            pltpu.async_copy(zt_hbm.at[sub.at[j]],
                             rows_v.at[pl.ds((half * _WR + j) * _LN, _LN)],
                             gsa)
            for j in range(_WR)
        ]

    def fire_s(w, half):
        sub = sidx_all.at[w]
        return [
            pltpu.async_copy(rows_v.at[pl.ds((half * _WR + j) * _LN, _LN)],
                             acc.at[sub.at[j]], ssa, add=True)
            for j in range(_WR)
        ]

    gds0 = fire_g(0, 0)
    for d in gds0:
        d.wait()
    plsc.subcore_barrier()

    def win(i, carry):
        p = lax.rem(i, 2)
        q = 1 - p

        @pl.when(i + 1 < nwin)
        def _():
            gds = fire_g(i + 1, q)
            sds = fire_s(i, p)
            for d in sds:
                d.wait()
            for d in gds:
                d.wait()

        @pl.when(i + 1 >= nwin)
        def _():
            sds = fire_s(i, p)
            for d in sds:
                d.wait()

        return carry

    lax.fori_loop(0, nwin, win, 0)
    plsc.subcore_barrier()
    pltpu.sync_copy(acc.at[pl.ds(s * per_sub, per_sub)],
                    out_hbm.at[pl.ds((c * _NS + s) * per_sub, per_sub)])


def kernel(x, edge_index, lambda_max, W0_0, W0_1, b0, W1_0, W1_1, b1, W2_0,
           W2_1, b2, Wf, bf):
    n, f_in = x.shape
    e = edge_index.shape[1]
    h = W0_0.shape[1]
    out_w = Wf.shape[1]
    f32 = jnp.float32

    rows = e // _LN
    rows_pad = ((rows + 2 * _NW * _WR - 1) // (2 * _NW * _WR)) * (2 * _NW * _WR)
    rows_per_w = rows_pad // _NW
    nwin = rows_per_w // _WR
    np_ = ((n + 2048 + _LN - 1) // _LN) * _LN
    per_sub = np_ // _NS

    row2 = edge_index[0].reshape(rows, _LN)
    col2 = edge_index[1].reshape(rows, _LN)
    lam2 = lambda_max.reshape(1, 1).astype(f32)
    zeros_h = jnp.zeros((per_sub, h), f32)
    zeros_d = jnp.zeros((per_sub, 8), f32)
    ones_d = jnp.ones((_LN, 8), f32)
    b0r = b0.reshape(1, h)
    b1r = b1.reshape(1, h)
    b2r = b2.reshape(1, h)
    bfr = bf.reshape(1, out_w)

    gidx, sidx, didx = pl.pallas_call(
        functools.partial(_prep_body, n=n, np_=np_, rows=rows,
                          rows_pad=rows_pad),
        out_shape=[jax.ShapeDtypeStruct((rows_pad, _LN), jnp.int32)] * 3,
    )(row2, col2)

    gidx3 = gidx.reshape(rows_pad // _WR, _WR, _LN)
    sidx3 = sidx.reshape(rows_pad // _WR, _WR, _LN)
    didx3 = didx.reshape(rows_pad // _WR, _WR, _LN)
    mesh = plsc.VectorSubcoreMesh(core_axis_name="c", subcore_axis_name="s")
    deg = pl.kernel(
        functools.partial(_sc_deg_body, np_=np_, per_sub=per_sub, nwin=nwin),
        out_type=jax.ShapeDtypeStruct((_NC * np_, 8), f32),
        mesh=mesh,
        compiler_params=_SC_PARAMS,
        scratch_types=[
            pltpu.VMEM((nwin, _WR, _LN), jnp.int32),
            pltpu.VMEM((_LN, 8), f32),
            pltpu.VMEM_SHARED((np_, 8), f32),
            pltpu.SemaphoreType.DMA,
        ],
    )(didx3, ones_d, zeros_d)

    y1p, z1 = pl.pallas_call(
        _proj0_body,
        out_shape=[jax.ShapeDtypeStruct((n, h), f32)] * 2,
    )(x, W0_0, W0_1, b0r, lam2)

    dis8, zt1 = pl.pallas_call(
        functools.partial(_dis_body, n=n, np_=np_),
        out_shape=[jax.ShapeDtypeStruct((n, 8), f32),
                   jax.ShapeDtypeStruct((n, h), f32)],
    )(deg, z1)

    spmv = pl.kernel(
        functools.partial(_sc_spmv_body, n=n, np_=np_, per_sub=per_sub,
                          nwin=nwin, h=h),
        out_type=jax.ShapeDtypeStruct((_NC * np_, h), f32),
        mesh=mesh,
        compiler_params=_SC_PARAMS,
        scratch_types=[
            pltpu.VMEM((nwin, _WR, _LN), jnp.int32),
            pltpu.VMEM((nwin, _WR, _LN), jnp.int32),
            pltpu.VMEM((2 * _WR * _LN, h), f32),
            pltpu.VMEM_SHARED((np_, h), f32),
            pltpu.SemaphoreType.DMA,
            pltpu.SemaphoreType.DMA,
        ],
    )

    mid = functools.partial(_mid_body, n=n, np_=np_)

    acc1 = spmv(gidx3, sidx3, zt1, zeros_h)
    y2p, zt2 = pl.pallas_call(
        mid,
        out_shape=[jax.ShapeDtypeStruct((n, h), f32)] * 2,
    )(acc1, dis8, y1p, W1_0, W1_1, b1r, lam2)

    acc2 = spmv(gidx3, sidx3, zt2, zeros_h)
    y3p, zt3 = pl.pallas_call(
        mid,
        out_shape=[jax.ShapeDtypeStruct((n, h), f32)] * 2,
    )(acc2, dis8, y2p, W2_0, W2_1, b2r, lam2)

    acc3 = spmv(gidx3, sidx3, zt3, zeros_h)
    out = pl.pallas_call(
        functools.partial(_final_body, n=n, np_=np_),
        out_shape=jax.ShapeDtypeStruct((n, out_w), f32),
    )(acc3, dis8, y3p, Wf, bfr, lam2)
    return out
